# baseline (speedup 1.0000x reference)
import jax
import jax.numpy as jnp
from jax.experimental import pallas as pl
from jax.experimental.pallas import tpu as pltpu

_S = 32
_D = 64
_N = 1025
_R = 5
_NQ = 4
_STEPS = 52
_QBASE = (0, 260, 520, 780)
_QSTEPS = (52, 52, 52, 49)


def _compute_rows(tv_ref, th_ref, buf, base):
    t0 = tv_ref[0:1, :] + th_ref[0:1, :]
    for r in range(_R):
        g = base + r
        gm = jnp.maximum(g - 1, 0)
        bi = gm // _S
        ci = gm - bi * _S
        vs = tv_ref[pl.ds(33 - bi, _S), :]
        hs = th_ref[pl.ds(33 - ci, _S), :]
        pat = (vs[:, None, :] + hs[None, :, :]).reshape(_S * _S, _D)
        buf[r, 0:1, :] = t0
        buf[r, 1:, :] = pat
    @pl.when(base == 0)
    def _():
        buf[0, :, :] = jnp.broadcast_to(t0, (_N, _D))


def _rp2d_body(tv_ref, th_ref, out_ref, *scratch):
    bufs = scratch[:_NQ]
    sems = scratch[_NQ:]
    s = pl.program_id(0)
    for q in range(_NQ):
        @pl.when((s >= 1) & (s <= _QSTEPS[q]))
        def _(q=q):
            pltpu.make_async_copy(
                bufs[q], out_ref.at[pl.ds(0, _R)], sems[q]).wait()
    for q in range(_NQ):
        _compute_rows(tv_ref, th_ref, bufs[q], _QBASE[q] + s * _R)
    for q in range(_NQ):
        @pl.when(s < _QSTEPS[q])
        def _(q=q):
            pltpu.make_async_copy(
                bufs[q],
                out_ref.at[pl.ds(_QBASE[q] + s * _R, _R)],
                sems[q]).start()
    @pl.when(s == _STEPS - 1)
    def _():
        for q in range(_NQ):
            @pl.when(_QSTEPS[q] == _STEPS)
            def _(q=q):
                pltpu.make_async_copy(
                    bufs[q], out_ref.at[pl.ds(0, _R)], sems[q]).wait()


def kernel(length_q, length_k, embeddings_table_v, embeddings_table_h):
    del length_q, length_k
    tv = jnp.pad(embeddings_table_v, ((0, 6), (0, 0)))
    th = jnp.pad(embeddings_table_h, ((0, 6), (0, 0)))
    return pl.pallas_call(
        _rp2d_body,
        grid=(_STEPS,),
        in_specs=[
            pl.BlockSpec((72, _D), lambda i: (0, 0)),
            pl.BlockSpec((72, _D), lambda i: (0, 0)),
        ],
        out_specs=pl.BlockSpec(memory_space=pl.ANY),
        out_shape=jax.ShapeDtypeStruct((_N, _N, _D), jnp.float32),
        scratch_shapes=(
            [pltpu.VMEM((_R, _N, _D), jnp.float32) for _ in range(_NQ)]
            + [pltpu.SemaphoreType.DMA for _ in range(_NQ)]
        ),
    )(tv, th)
